# Initial kernel scaffold; baseline (speedup 1.0000x reference)
import jax
import jax.numpy as jnp
from jax.experimental import pallas as pl


def kernel(input_ids, segment_ids, tok_table, pos_table, seg_table, gamma, beta):
    raise NotImplementedError("write your pallas kernel here")


# speedup vs baseline: 1.2876x; 1.2876x over previous
import functools

import jax
import jax.numpy as jnp
from jax import lax
from jax.experimental import pallas as pl
from jax.experimental.pallas import tpu as pltpu
from jax.experimental.pallas import tpu_sc as plsc

D_MODEL = 768
N_SEQ = 2048
N_BATCH = 4
LANES = 16
NJ = D_MODEL // LANES
GRP = 4
NC, NS = 2, 16
NW = NC * NS
POS_PER_W = N_SEQ // NW
CH = 32
CHUNKS_PER_BATCH = POS_PER_W // CH
N_CHUNKS = N_BATCH * CHUNKS_PER_BATCH
TOK_PER_W = N_CHUNKS * CH
N_TOK = N_BATCH * N_SEQ
EPS = 1e-12
NBUF = 3


def _allsum(x, iota):
    for k in (8, 4, 2, 1):
        x = x + x.at[jnp.bitwise_xor(iota, k)].get(mode="promise_in_bounds")
    return x


def _rsqrt_vec(x):
    i = lax.bitcast_convert_type(x, jnp.int32)
    i = jnp.int32(0x5F3759DF) - lax.shift_right_logical(i, 1)
    y = lax.bitcast_convert_type(i, jnp.float32)
    for _ in range(2):
        y = y * (1.5 - 0.5 * x * y * y)
    return y


def _sc_body(use_gb, tok_hbm, ids_hbm, sid_hbm, pos_hbm, seg_hbm, gam_hbm,
             bet_hbm, out_hbm, pos_v, tok_a, tok_b, tok_c, seg_v, gam_v,
             bet_v, idx_v, sid_v, gsem_a, gsem_b, gsem_c, osem_a, osem_b,
             osem_c):
    wid = lax.axis_index("s") * NC + lax.axis_index("c")
    pbase = pl.multiple_of(wid * POS_PER_W, POS_PER_W)

    pltpu.sync_copy(pos_hbm.at[pl.ds(pbase, POS_PER_W)], pos_v)
    pltpu.sync_copy(seg_hbm, seg_v)
    if use_gb:
        pltpu.sync_copy(gam_hbm, gam_v)
        pltpu.sync_copy(bet_hbm, bet_v)
    wbase = pl.multiple_of(wid * TOK_PER_W, TOK_PER_W)
    pltpu.sync_copy(ids_hbm.at[pl.ds(wbase, TOK_PER_W)], idx_v)
    pltpu.sync_copy(sid_hbm.at[pl.ds(wbase, TOK_PER_W)],
                    sid_v.at[pl.ds(0, TOK_PER_W)])
    iota = lax.iota(jnp.int32, LANES)

    tok_bufs = [tok_a, tok_b, tok_c]
    gsems = [gsem_a, gsem_b, gsem_c]
    osems = [osem_a, osem_b, osem_c]

    def tok_base(c):
        b, h = divmod(c, CHUNKS_PER_BATCH)
        return pl.multiple_of(b * N_SEQ + pbase + h * CH, CH)

    def start_gather(c):
        s = c % NBUF
        return pltpu.async_copy(
            tok_hbm.at[idx_v.at[pl.ds(c * CH, CH)]], tok_bufs[s], gsems[s])

    pending_g = {0: start_gather(0), 1: start_gather(1)}
    pending_o = {}

    for c in range(N_CHUNKS):
        s = c % NBUF
        if c + 2 < N_CHUNKS:
            if c - 1 in pending_o:
                pending_o.pop(c - 1).wait()
            pending_g[c + 2] = start_gather(c + 2)
        pending_g.pop(c).wait()

        tok_v = tok_bufs[s]
        prow0 = (c % CHUNKS_PER_BATCH) * CH
        sid0 = c * CH

        def token_body(t, carry, tok_v=tok_v, prow0=prow0, sid0=sid0):
            srow = sid_v[pl.ds(sid0 + t, LANES)][0]
            accs = [jnp.zeros((LANES,), jnp.float32) for _ in range(GRP)]
            accs2 = [jnp.zeros((LANES,), jnp.float32) for _ in range(GRP)]
            for j0 in range(0, NJ, GRP):
                sls = [pl.ds((j0 + i) * LANES, LANES) for i in range(GRP)]
                tv = [tok_v[t, sl] for sl in sls]
                pv = [pos_v[prow0 + t, sl] for sl in sls]
                sv = [seg_v[srow, sl] for sl in sls]
                u = [a + b for a, b in zip(tv, pv)]
                v = [a + b for a, b in zip(u, sv)]
                for i in range(GRP):
                    tok_v[t, sls[i]] = v[i]
                sq = [a * a for a in v]
                accs = [a + b for a, b in zip(accs, v)]
                accs2 = [a + b for a, b in zip(accs2, sq)]

            acc = (accs[0] + accs[1]) + (accs[2] + accs[3])
            acc2 = (accs2[0] + accs2[1]) + (accs2[2] + accs2[3])
            mean = _allsum(acc, iota) * (1.0 / D_MODEL)
            ex2 = _allsum(acc2, iota) * (1.0 / D_MODEL)
            var = jnp.maximum(ex2 - mean * mean, 0.0)
            rstd = _rsqrt_vec(var + EPS)
            mr = mean * rstd

            for j0 in range(0, NJ, GRP):
                sls = [pl.ds((j0 + i) * LANES, LANES) for i in range(GRP)]
                v = [tok_v[t, sl] for sl in sls]
                o = [a * rstd - mr for a in v]
                if use_gb:
                    g = [gam_v[sl] for sl in sls]
                    bta = [bet_v[sl] for sl in sls]
                    o = [a * gg + bb for a, gg, bb in zip(o, g, bta)]
                for i in range(GRP):
                    tok_v[t, sls[i]] = o[i]
            return carry

        lax.fori_loop(0, CH, token_body, 0)
        pending_o[c] = pltpu.async_copy(
            tok_v, out_hbm.at[pl.ds(tok_base(c), CH)], osems[s])

    for c in sorted(pending_o):
        pending_o.pop(c).wait()


def _make_kernel(use_gb):
    return pl.kernel(
        functools.partial(_sc_body, use_gb),
        out_type=jax.ShapeDtypeStruct((N_TOK, D_MODEL), jnp.float32),
        mesh=plsc.VectorSubcoreMesh(core_axis_name="c", subcore_axis_name="s"),
        scratch_types=[
            pltpu.VMEM((POS_PER_W, D_MODEL), jnp.float32),
            pltpu.VMEM((CH, D_MODEL), jnp.float32),
            pltpu.VMEM((CH, D_MODEL), jnp.float32),
            pltpu.VMEM((CH, D_MODEL), jnp.float32),
            pltpu.VMEM((2, D_MODEL), jnp.float32),
            pltpu.VMEM((D_MODEL,), jnp.float32),
            pltpu.VMEM((D_MODEL,), jnp.float32),
            pltpu.VMEM((TOK_PER_W,), jnp.int32),
            pltpu.VMEM((TOK_PER_W + LANES,), jnp.int32),
            pltpu.SemaphoreType.DMA,
            pltpu.SemaphoreType.DMA,
            pltpu.SemaphoreType.DMA,
            pltpu.SemaphoreType.DMA,
            pltpu.SemaphoreType.DMA,
            pltpu.SemaphoreType.DMA,
        ],
    )


@jax.jit
def _sc_embed_ln(tok_table, ids, sids, pos_table, seg_table, gamma, beta):
    plain = jnp.logical_and(jnp.all(gamma == 1.0), jnp.all(beta == 0.0))
    args = (tok_table, ids, sids, pos_table, seg_table, gamma, beta)
    return lax.cond(plain,
                    lambda *a: _make_kernel(False)(*a),
                    lambda *a: _make_kernel(True)(*a),
                    *args)


def _worker_major(x):
    return (x.reshape(N_BATCH, NW, POS_PER_W).transpose(1, 0, 2)
            .reshape(-1).astype(jnp.int32))


def kernel(input_ids, segment_ids, tok_table, pos_table, seg_table, gamma, beta):
    b, s = input_ids.shape
    ids = _worker_major(input_ids)
    sids = _worker_major(segment_ids)
    out = _sc_embed_ln(tok_table, ids, sids, pos_table, seg_table, gamma, beta)
    return out.reshape(b, s, D_MODEL)
